# baseline (speedup 1.0000x reference)
import functools

import jax
import jax.numpy as jnp
from jax import lax
from jax.experimental import pallas as pl
from jax.experimental.pallas import tpu as pltpu
from jax.experimental.pallas import tpu_sc as plsc

N = 100000
E = 3200000
FILTERS = 16
OUT_LEN = 14

NC = 2
NS = 16
NPT = 6272
N_PAD = NS * NPT
EPC = E // NC
EPW = E // (NC * NS)
CHUNK = 4000
NCHUNK = EPW // CHUNK
NBUF = 3
BN = 2048
NBLK = N_PAD // BN
BNO = 2048
NBLKO = (N + BNO - 1) // BNO

_mesh = plsc.VectorSubcoreMesh(core_axis_name="c", subcore_axis_name="s")
_HP = lax.Precision.HIGHEST


def _sc_degree_body(ei_hbm, ew_hbm, z1_hbm, deg_out,
                    didx0, didx1, didx2, ewb0, ewb1, ewb2,
                    insem0, insem1, insem2, scsem, deg_sh):
    didx = (didx0, didx1, didx2)
    ewb = (ewb0, ewb1, ewb2)
    insem = (insem0, insem1, insem2)
    cid = lax.axis_index("c")
    sid = lax.axis_index("s")
    sl = pl.ds(sid * NPT, NPT)
    pltpu.sync_copy(z1_hbm.at[sl], deg_sh.at[sl])
    plsc.subcore_barrier()
    ebase = cid * EPC + sid * EPW

    def start_in(k):
        b = k % NBUF
        base = ebase + k * CHUNK
        return [
            pltpu.async_copy(ei_hbm.at[pl.ds(E + base, CHUNK)], didx[b],
                             insem[b]),
            pltpu.async_copy(ew_hbm.at[pl.ds(base, CHUNK)], ewb[b],
                             insem[b]),
        ]

    ind = {0: start_in(0), 1: start_in(1)}
    scd = {}
    for k in range(NCHUNK):
        b = k % NBUF
        for d in ind.pop(k):
            d.wait()
        if k - 1 in scd:
            scd.pop(k - 1).wait()
        scd[k] = pltpu.async_copy(ewb[b], deg_sh.at[didx[b]],
                                  scsem, add=True)
        if k + 2 < NCHUNK:
            ind[k + 2] = start_in(k + 2)
    scd.pop(NCHUNK - 1).wait()
    plsc.subcore_barrier()
    pltpu.sync_copy(deg_sh.at[sl], deg_out.at[cid, sl])


def _sc_messages_body(ei_hbm, ew_hbm, yp_hbm, z1_hbm, acc_out,
                      sidx0, sidx1, sidx2, didx0, didx1, didx2,
                      ewb0, ewb1, ewb2, ypg, r0, r1,
                      insem0, insem1, insem2, scsem,
                      yp_sh, a0_sh, a1_sh):
    sidx = (sidx0, sidx1, sidx2)
    didx = (didx0, didx1, didx2)
    ewb = (ewb0, ewb1, ewb2)
    insem = (insem0, insem1, insem2)
    cid = lax.axis_index("c")
    sid = lax.axis_index("s")
    sl = pl.ds(sid * NPT, NPT)
    pltpu.sync_copy(yp_hbm.at[sl], yp_sh.at[sl])
    pltpu.sync_copy(z1_hbm.at[sl], a0_sh.at[sl])
    pltpu.sync_copy(z1_hbm.at[sl], a1_sh.at[sl])
    plsc.subcore_barrier()
    ebase = cid * EPC + sid * EPW

    def start_in(k):
        b = k % NBUF
        base = ebase + k * CHUNK
        return [
            pltpu.async_copy(ei_hbm.at[pl.ds(base, CHUNK)], sidx[b],
                             insem[b]),
            pltpu.async_copy(ei_hbm.at[pl.ds(E + base, CHUNK)], didx[b],
                             insem[b]),
            pltpu.async_copy(ew_hbm.at[pl.ds(base, CHUNK)], ewb[b],
                             insem[b]),
        ]

    ind = {0: start_in(0), 1: start_in(1)}
    scd = {}
    for k in range(NCHUNK):
        b = k % NBUF
        rb0, rb1 = (r0, r1) if k % 2 == 0 else (r1, r0)
        for d in ind.pop(k):
            d.wait()
        pltpu.sync_copy(yp_sh.at[sidx[b]], ypg)
        if k - 1 in scd:
            for d in scd.pop(k - 1):
                d.wait()

        def mul(j, c):
            v16 = pl.ds(j * 16, 16)
            packed = plsc.bitcast(ypg[v16], jnp.bfloat16)
            y0, y1 = plsc.unpack(packed, format=plsc.PackFormat.INTERLEAVED)
            w = ewb[b][v16]
            rb0[v16] = y0 * w
            rb1[v16] = y1 * w
            return c

        lax.fori_loop(0, CHUNK // 16, mul, 0)
        scd[k] = [
            pltpu.async_copy(rb0, a0_sh.at[didx[b]], scsem, add=True),
            pltpu.async_copy(rb1, a1_sh.at[didx[b]], scsem, add=True),
        ]
        if k + 2 < NCHUNK:
            ind[k + 2] = start_in(k + 2)
    for d in scd.pop(NCHUNK - 1):
        d.wait()
    plsc.subcore_barrier()
    row = cid * 2
    pltpu.sync_copy(a0_sh.at[sl], acc_out.at[row, sl])
    pltpu.sync_copy(a1_sh.at[sl], acc_out.at[row + 1, sl])


def _make_sc_kernels(interpret=False):
    deg = pl.kernel(
        _sc_degree_body,
        out_type=jax.ShapeDtypeStruct((NC, N_PAD), jnp.float32),
        mesh=_mesh,
        scratch_types=(
            [pltpu.VMEM((CHUNK,), jnp.int32)] * NBUF
            + [pltpu.VMEM((CHUNK,), jnp.float32)] * NBUF
            + [pltpu.SemaphoreType.DMA] * NBUF
            + [
                pltpu.SemaphoreType.DMA,
                pltpu.VMEM_SHARED((N_PAD,), jnp.float32),
            ]
        ),
        interpret=interpret,
    )
    msg = pl.kernel(
        _sc_messages_body,
        out_type=jax.ShapeDtypeStruct((NC * 2, N_PAD), jnp.float32),
        mesh=_mesh,
        compiler_params=pltpu.CompilerParams(needs_layout_passes=False),
        scratch_types=(
            [pltpu.VMEM((CHUNK,), jnp.int32)] * (2 * NBUF)
            + [pltpu.VMEM((CHUNK,), jnp.float32)] * (NBUF + 3)
            + [pltpu.SemaphoreType.DMA] * (NBUF + 1)
            + [pltpu.VMEM_SHARED((N_PAD,), jnp.float32)] * 3
        ),
        interpret=interpret,
    )
    return deg, msg


_sc_degree, _sc_messages = _make_sc_kernels()


def _tc_norm_body(x_ref, pos_ref, degp_ref, wp_ref, bpos_ref,
                  y_ref, dinv_ref, ypack_ref):
    xb = jnp.nan_to_num(x_ref[...])
    xf = xb + jnp.dot(pos_ref[...], wp_ref[...], precision=_HP,
                      preferred_element_type=jnp.float32) + bpos_ref[...]
    degp = degp_ref[...]
    deg = degp[0:1, :] + degp[1:2, :] + 1.0
    dinv = jnp.where(deg > 0, lax.rsqrt(deg), 0.0)
    dinv_col = jnp.transpose(dinv)
    y = xf * dinv_col
    y_ref[...] = y
    dinv_ref[...] = dinv_col
    yb = y.astype(jnp.bfloat16)
    u = lax.bitcast_convert_type(yb, jnp.uint16).astype(jnp.uint32)
    w = jnp.bitwise_or(u[:, 0:1], jnp.left_shift(u[:, 1:2], 16))
    ypack_ref[...] = lax.bitcast_convert_type(w, jnp.float32)


def _tc_norm(x_p, pos_p, deg_part, w_pos, b_pos_row):
    return pl.pallas_call(
        _tc_norm_body,
        grid=(NBLK,),
        in_specs=[
            pl.BlockSpec((BN, 2), lambda i: (i, 0)),
            pl.BlockSpec((BN, 9), lambda i: (i, 0)),
            pl.BlockSpec((2, BN), lambda i: (0, i)),
            pl.BlockSpec((9, 2), lambda i: (0, 0)),
            pl.BlockSpec((1, 2), lambda i: (0, 0)),
        ],
        out_specs=[
            pl.BlockSpec((BN, 2), lambda i: (i, 0)),
            pl.BlockSpec((BN, 1), lambda i: (i, 0)),
            pl.BlockSpec((BN, 1), lambda i: (i, 0)),
        ],
        out_shape=[
            jax.ShapeDtypeStruct((N_PAD, 2), jnp.float32),
            jax.ShapeDtypeStruct((N_PAD, 1), jnp.float32),
            jax.ShapeDtypeStruct((N_PAD, 1), jnp.float32),
        ],
    )(x_p, pos_p, deg_part, w_pos, b_pos_row)


def _tc_out_body(acc_ref, y_ref, dinv_ref, consts_ref, wout_ref, bout_ref, out_ref):
    a = acc_ref[...]
    y = y_ref[...]
    dinv = dinv_ref[...]
    c = consts_ref[...]
    H = jnp.zeros((BNO, FILTERS), dtype=jnp.float32)
    for p in range(2):
        arow = jnp.reshape(a[0:1, p:p + 1, :] + a[1:2, p:p + 1, :], (1, BNO))
        sp = (jnp.transpose(arow) + y[:, p:p + 1]) * dinv
        Asig = jnp.dot(sp, c[0:1, :], precision=_HP,
                       preferred_element_type=jnp.float32)
        Atan = jnp.dot(sp, c[2:3, :], precision=_HP,
                       preferred_element_type=jnp.float32)
        G = jax.nn.sigmoid(Asig + c[1:2, :])
        T = jnp.tanh(Atan + c[3:4, :])
        H = H + c[4 + p:5 + p, :] * (1.0 - G) * T
    h = jnp.maximum(H, 0.0)
    out_ref[...] = (jnp.dot(h, wout_ref[...], precision=_HP,
                            preferred_element_type=jnp.float32)
                    + bout_ref[...])


def _tc_out(acc_part, y_n, dinv, consts, w_out, b_out_row):
    return pl.pallas_call(
        _tc_out_body,
        grid=(NBLKO,),
        in_specs=[
            pl.BlockSpec((2, 2, BNO), lambda i: (0, 0, i)),
            pl.BlockSpec((BNO, 2), lambda i: (i, 0)),
            pl.BlockSpec((BNO, 1), lambda i: (i, 0)),
            pl.BlockSpec((6, FILTERS), lambda i: (0, 0)),
            pl.BlockSpec((FILTERS, OUT_LEN), lambda i: (0, 0)),
            pl.BlockSpec((1, OUT_LEN), lambda i: (0, 0)),
        ],
        out_specs=pl.BlockSpec((BNO, OUT_LEN), lambda i: (i, 0)),
        out_shape=jax.ShapeDtypeStruct((N, OUT_LEN), jnp.float32),
    )(acc_part, y_n, dinv, consts, w_out, b_out_row)


def kernel(x, edge_index, edge_weight, pos_src, W_pos, b_pos, attention,
           W_z, b_z, Lw_z, Lb_z, W_r, b_r, Lw_r, Lb_r, W_h, b_h, Lw_h, Lb_h,
           W_out, b_out):
    pad = N_PAD - N
    x_p = jnp.pad(x, ((0, pad), (0, 0)))
    pos_p = jnp.pad(pos_src, ((0, pad), (0, 0)))
    zeros_n = jnp.zeros((N_PAD,), jnp.float32)

    az = (W_z @ Lw_z[:FILTERS])[0]
    cz = b_z @ Lw_z[:FILTERS] + Lb_z
    ah = (W_h @ Lw_h[:FILTERS])[0]
    ch = b_h @ Lw_h[:FILTERS] + Lb_h
    probs = jax.nn.softmax(attention, axis=0)
    consts = jnp.stack([
        az, cz, ah, ch,
        jnp.full((FILTERS,), 1.0, jnp.float32) * probs[0],
        jnp.full((FILTERS,), 1.0, jnp.float32) * probs[1],
    ])

    ei_flat = edge_index.reshape(2 * E)
    deg_part = _sc_degree(ei_flat, edge_weight, zeros_n)
    y_n, dinv, ypack = _tc_norm(x_p, pos_p, deg_part, W_pos, b_pos[None, :])
    acc_part = _sc_messages(ei_flat, edge_weight, ypack[:, 0],
                            zeros_n)
    acc_part = acc_part.reshape(NC, 2, N_PAD)
    out = _tc_out(acc_part, y_n, dinv, consts, W_out, b_out[None, :])
    return (out,)


# speedup vs baseline: 2.2218x; 2.2218x over previous
import functools

import jax
import jax.numpy as jnp
from jax import lax
from jax.experimental import pallas as pl
from jax.experimental.pallas import tpu as pltpu
from jax.experimental.pallas import tpu_sc as plsc

N = 100000
E = 3200000
FILTERS = 16
OUT_LEN = 14

NC = 2
NS = 16
NPT = 6272
N_PAD = NS * NPT
EPC = E // NC
EPW = E // (NC * NS)
CHUNK = 4000
NCHUNK = EPW // CHUNK
NBUF = 3
BN = 2048
NBLK = N_PAD // BN
BNO = 2048
NBLKO = (N + BNO - 1) // BNO

_mesh = plsc.VectorSubcoreMesh(core_axis_name="c", subcore_axis_name="s")
_HP = lax.Precision.HIGHEST


def _sc_degree_body(ei_hbm, ew_hbm, z1_hbm, deg_out,
                    didx0, didx1, didx2, ewb0, ewb1, ewb2,
                    insem0, insem1, insem2, scsem, deg_sh):
    didx = (didx0, didx1, didx2)
    ewb = (ewb0, ewb1, ewb2)
    insem = (insem0, insem1, insem2)
    cid = lax.axis_index("c")
    sid = lax.axis_index("s")
    sl = pl.ds(sid * NPT, NPT)
    pltpu.sync_copy(z1_hbm.at[sl], deg_sh.at[sl])
    plsc.subcore_barrier()
    ebase = cid * EPC + sid * EPW

    def start_in(k):
        b = k % NBUF
        base = ebase + k * CHUNK
        return [
            pltpu.async_copy(ei_hbm.at[pl.ds(E + base, CHUNK)], didx[b],
                             insem[b]),
            pltpu.async_copy(ew_hbm.at[pl.ds(base, CHUNK)], ewb[b],
                             insem[b]),
        ]

    ind = {0: start_in(0), 1: start_in(1)}
    scd = {}
    for k in range(NCHUNK):
        b = k % NBUF
        for d in ind.pop(k):
            d.wait()
        if k - 1 in scd:
            scd.pop(k - 1).wait()
        scd[k] = pltpu.async_copy(ewb[b], deg_sh.at[didx[b]],
                                  scsem, add=True)
        if k + 2 < NCHUNK:
            ind[k + 2] = start_in(k + 2)
    scd.pop(NCHUNK - 1).wait()
    plsc.subcore_barrier()
    pltpu.sync_copy(deg_sh.at[sl], deg_out.at[cid, sl])


def _sc_messages_body(ei_hbm, ew_hbm, yp_hbm, z1_hbm, acc_out,
                      sidx0, sidx1, sidx2, didx0, didx1, didx2,
                      ewb0, ewb1, ewb2, ypg, r0, r1,
                      insem0, insem1, insem2, scsem,
                      yp_sh, a0_sh, a1_sh):
    sidx = (sidx0, sidx1, sidx2)
    didx = (didx0, didx1, didx2)
    ewb = (ewb0, ewb1, ewb2)
    insem = (insem0, insem1, insem2)
    cid = lax.axis_index("c")
    sid = lax.axis_index("s")
    sl = pl.ds(sid * NPT, NPT)
    pltpu.sync_copy(yp_hbm.at[sl], yp_sh.at[sl])
    pltpu.sync_copy(z1_hbm.at[sl], a0_sh.at[sl])
    pltpu.sync_copy(z1_hbm.at[sl], a1_sh.at[sl])
    plsc.subcore_barrier()
    ebase = cid * EPC + sid * EPW

    def start_in(k):
        b = k % NBUF
        base = ebase + k * CHUNK
        return [
            pltpu.async_copy(ei_hbm.at[pl.ds(base, CHUNK)], sidx[b],
                             insem[b]),
            pltpu.async_copy(ei_hbm.at[pl.ds(E + base, CHUNK)], didx[b],
                             insem[b]),
            pltpu.async_copy(ew_hbm.at[pl.ds(base, CHUNK)], ewb[b],
                             insem[b]),
        ]

    ind = {0: start_in(0), 1: start_in(1)}
    scd = {}
    for k in range(NCHUNK):
        b = k % NBUF
        rb0, rb1 = (r0, r1) if k % 2 == 0 else (r1, r0)
        for d in ind.pop(k):
            d.wait()
        pltpu.sync_copy(yp_sh.at[sidx[b]], ypg)
        if k - 1 in scd:
            for d in scd.pop(k - 1):
                d.wait()

        def mul(j, c):
            v16 = pl.ds(j * 16, 16)
            packed = plsc.bitcast(ypg[v16], jnp.bfloat16)
            y0, y1 = plsc.unpack(packed, format=plsc.PackFormat.INTERLEAVED)
            w = ewb[b][v16]
            rb0[v16] = y0 * w
            rb1[v16] = y1 * w
            return c

        lax.fori_loop(0, CHUNK // 16, mul, 0)
        scd[k] = [
            pltpu.async_copy(rb0, a0_sh.at[didx[b]], scsem, add=True),
            pltpu.async_copy(rb1, a1_sh.at[didx[b]], scsem, add=True),
        ]
        if k + 2 < NCHUNK:
            ind[k + 2] = start_in(k + 2)
    for d in scd.pop(NCHUNK - 1):
        d.wait()
    plsc.subcore_barrier()
    row = cid * 2
    pltpu.sync_copy(a0_sh.at[sl], acc_out.at[row, sl])
    pltpu.sync_copy(a1_sh.at[sl], acc_out.at[row + 1, sl])


def _make_sc_kernels(interpret=False):
    deg = pl.kernel(
        _sc_degree_body,
        out_type=jax.ShapeDtypeStruct((NC, N_PAD), jnp.float32),
        mesh=_mesh,
        scratch_types=(
            [pltpu.VMEM((CHUNK,), jnp.int32)] * NBUF
            + [pltpu.VMEM((CHUNK,), jnp.float32)] * NBUF
            + [pltpu.SemaphoreType.DMA] * NBUF
            + [
                pltpu.SemaphoreType.DMA,
                pltpu.VMEM_SHARED((N_PAD,), jnp.float32),
            ]
        ),
        interpret=interpret,
    )
    msg = pl.kernel(
        _sc_messages_body,
        out_type=jax.ShapeDtypeStruct((NC * 2, N_PAD), jnp.float32),
        mesh=_mesh,
        compiler_params=pltpu.CompilerParams(needs_layout_passes=False),
        scratch_types=(
            [pltpu.VMEM((CHUNK,), jnp.int32)] * (2 * NBUF)
            + [pltpu.VMEM((CHUNK,), jnp.float32)] * (NBUF + 3)
            + [pltpu.SemaphoreType.DMA] * (NBUF + 1)
            + [pltpu.VMEM_SHARED((N_PAD,), jnp.float32)] * 3
        ),
        interpret=interpret,
    )
    return deg, msg


_sc_degree, _sc_messages = _make_sc_kernels()


def _tc_norm_body(xt_ref, post_ref, degp_ref, wpt_ref, bpos_ref,
                  y_ref, dinv_ref, ypack_ref):
    xb = jnp.nan_to_num(xt_ref[...])
    xf = xb + jnp.dot(wpt_ref[...], post_ref[...],
                      preferred_element_type=jnp.float32) + bpos_ref[...]
    degp = degp_ref[...]
    deg = degp[0:1, :] + degp[1:2, :] + 1.0
    dinv = jnp.where(deg > 0, lax.rsqrt(deg), 0.0)
    y = xf * dinv
    y_ref[...] = y
    dinv_ref[...] = dinv
    yb = y.astype(jnp.bfloat16)
    u = lax.bitcast_convert_type(yb, jnp.uint16).astype(jnp.uint32)
    w = jnp.bitwise_or(u[0:1, :], jnp.left_shift(u[1:2, :], 16))
    ypack_ref[...] = lax.bitcast_convert_type(w, jnp.float32)


def _tc_norm(x_t, pos_t, deg_part, w_pos_t, b_pos_col):
    return pl.pallas_call(
        _tc_norm_body,
        grid=(NBLK,),
        in_specs=[
            pl.BlockSpec((2, BN), lambda i: (0, i)),
            pl.BlockSpec((9, BN), lambda i: (0, i)),
            pl.BlockSpec((2, BN), lambda i: (0, i)),
            pl.BlockSpec((2, 9), lambda i: (0, 0)),
            pl.BlockSpec((2, 1), lambda i: (0, 0)),
        ],
        out_specs=[
            pl.BlockSpec((2, BN), lambda i: (0, i)),
            pl.BlockSpec((1, BN), lambda i: (0, i)),
            pl.BlockSpec((1, BN), lambda i: (0, i)),
        ],
        out_shape=[
            jax.ShapeDtypeStruct((2, N_PAD), jnp.float32),
            jax.ShapeDtypeStruct((1, N_PAD), jnp.float32),
            jax.ShapeDtypeStruct((1, N_PAD), jnp.float32),
        ],
    )(x_t, pos_t, deg_part, w_pos_t, b_pos_col)


def _tc_out_body(acc_ref, y_ref, dinv_ref, consts_ref, wout_ref, bout_ref, out_ref):
    a = acc_ref[...]
    y = y_ref[...]
    dinv = dinv_ref[...]
    c = consts_ref[...]
    dn = (((0,), (0,)), ((), ()))
    H = jnp.zeros((BNO, FILTERS), dtype=jnp.float32)
    for p in range(2):
        sp = (a[p:p + 1, :] + a[2 + p:3 + p, :] + y[p:p + 1, :]) * dinv
        Asig = lax.dot_general(sp, c[0:1, :], dn,
                               preferred_element_type=jnp.float32)
        Atan = lax.dot_general(sp, c[2:3, :], dn,
                               preferred_element_type=jnp.float32)
        G = jax.nn.sigmoid(Asig + c[1:2, :])
        T = jnp.tanh(Atan + c[3:4, :])
        H = H + c[4 + p:5 + p, :] * (1.0 - G) * T
    h = jnp.maximum(H, 0.0)
    out_ref[...] = (jnp.dot(h, wout_ref[...],
                            preferred_element_type=jnp.float32)
                    + bout_ref[...])


def _tc_out(acc_part, y_cm, dinv, consts, w_out, b_out_row):
    return pl.pallas_call(
        _tc_out_body,
        grid=(NBLKO,),
        in_specs=[
            pl.BlockSpec((4, BNO), lambda i: (0, i)),
            pl.BlockSpec((2, BNO), lambda i: (0, i)),
            pl.BlockSpec((1, BNO), lambda i: (0, i)),
            pl.BlockSpec((6, FILTERS), lambda i: (0, 0)),
            pl.BlockSpec((FILTERS, OUT_LEN), lambda i: (0, 0)),
            pl.BlockSpec((1, OUT_LEN), lambda i: (0, 0)),
        ],
        out_specs=pl.BlockSpec((BNO, OUT_LEN), lambda i: (i, 0)),
        out_shape=jax.ShapeDtypeStruct((N, OUT_LEN), jnp.float32),
    )(acc_part, y_cm, dinv, consts, w_out, b_out_row)


def kernel(x, edge_index, edge_weight, pos_src, W_pos, b_pos, attention,
           W_z, b_z, Lw_z, Lb_z, W_r, b_r, Lw_r, Lb_r, W_h, b_h, Lw_h, Lb_h,
           W_out, b_out):
    pad = N_PAD - N
    x_t = jnp.pad(x, ((0, pad), (0, 0))).T
    pos_t = jnp.pad(pos_src, ((0, pad), (0, 0))).T
    zeros_n = jnp.zeros((N_PAD,), jnp.float32)

    az = (W_z @ Lw_z[:FILTERS])[0]
    cz = b_z @ Lw_z[:FILTERS] + Lb_z
    ah = (W_h @ Lw_h[:FILTERS])[0]
    ch = b_h @ Lw_h[:FILTERS] + Lb_h
    probs = jax.nn.softmax(attention, axis=0)
    consts = jnp.stack([
        az, cz, ah, ch,
        jnp.full((FILTERS,), 1.0, jnp.float32) * probs[0],
        jnp.full((FILTERS,), 1.0, jnp.float32) * probs[1],
    ])

    ei_flat = edge_index.reshape(2 * E)
    deg_part = _sc_degree(ei_flat, edge_weight, zeros_n)
    y_cm, dinv, ypack = _tc_norm(x_t, pos_t, deg_part, W_pos.T,
                                 b_pos[:, None])
    acc_part = _sc_messages(ei_flat, edge_weight, ypack.reshape(N_PAD),
                            zeros_n)
    out = _tc_out(acc_part, y_cm, dinv, consts, W_out, b_out[None, :])
    return (out,)
